# baseline (speedup 1.0000x reference)
import functools

import jax
import jax.numpy as jnp
from jax import lax
from jax.experimental import pallas as pl
from jax.experimental.pallas import tpu as pltpu
from jax.experimental.pallas import tpu_sc as plsc

N_TOKENS = 1000
BATCH = 16384
SEQ_LEN = 200
OUT_COLS = N_TOKENS - 1

_INFO = plsc.get_sparse_core_info()
NUM_CORES = _INFO.num_cores
NUM_SUBCORES = _INFO.num_subcores
LANES = _INFO.num_lanes
NW = NUM_CORES * NUM_SUBCORES

STRIPE = 128
SPW = BATCH // (NW * STRIPE)
SEQ_TILES = SEQ_LEN // 8
HALF0 = 496
HALF1 = OUT_COLS - HALF0
TILE_GROUPS = 8 * STRIPE // LANES


def _bow_body(in_hbm, z_hbm, out_hbm, ina, inb, cnt0, cnt1,
              in_sem_a, in_sem_b, out_sem0, out_sem1, zf_sem0, zf_sem1):
    wid = lax.axis_index("s") * NUM_CORES + lax.axis_index("c")
    iota = lax.iota(jnp.int32, LANES)
    zeros = jnp.zeros((LANES,), jnp.float32)
    ones = jnp.ones((LANES,), jnp.float32)

    cnts = (cnt0, cnt1)
    out_sems = (out_sem0, out_sem1)
    zf_sems = (zf_sem0, zf_sem1)
    halves = ((0, HALF0), (HALF0, HALF1))
    base = wid * SPW * STRIPE

    def make_zero(cnt):
        def zero_step(j):
            cnt[j >> 3, pl.ds((j & 7) * LANES, LANES)] = zeros
        return zero_step

    def make_scatter(inb_, cnt, r0, nrows):
        lo = r0 + 1
        hi = r0 + nrows

        def tok_step(g):
            k = (g & 7) * LANES
            toks = inb_[g >> 3, pl.ds(k, LANES)]
            mask = (toks >= lo) & (toks <= hi)
            plsc.addupdate_scatter(
                cnt, [toks - lo, k + iota], ones, mask=mask)
        return tok_step

    def enqueue_fill(h):
        pltpu.async_copy(
            z_hbm.at[pl.ds(0, halves[h][1])], cnts[h], zf_sems[h])

    def wait_fill(h):
        pltpu.make_async_copy(
            z_hbm.at[pl.ds(0, halves[h][1])], cnts[h], zf_sems[h]).wait()

    plsc.parallel_loop(0, HALF0 * 8, unroll=12)(make_zero(cnt0))
    enqueue_fill(1)

    out_cp = {}
    for u in range(SPW * 2):
        s, h = u // 2, u % 2
        r0, nrows = halves[h]
        cnt = cnts[h]
        col = pl.ds(base + s * STRIPE, STRIPE)

        def start(t, buf, sem):
            return pltpu.async_copy(
                in_hbm.at[pl.ds(t * 8, 8), col], buf, sem)

        def wait(buf, sem):
            pltpu.make_async_copy(
                in_hbm.at[pl.ds(0, 8), col], buf, sem).wait()

        if u >= 1:
            wait_fill(h)

        scat_a = make_scatter(ina, cnt, r0, nrows)
        scat_b = make_scatter(inb, cnt, r0, nrows)
        start(0, ina, in_sem_a)

        def tile_pair(i):
            t = i * 2
            start(t + 1, inb, in_sem_b)
            wait(ina, in_sem_a)
            plsc.parallel_loop(0, TILE_GROUPS, unroll=8)(scat_a)
            start(t + 2, ina, in_sem_a)
            wait(inb, in_sem_b)
            plsc.parallel_loop(0, TILE_GROUPS, unroll=8)(scat_b)

        pl.loop(0, 6)(tile_pair)
        if u >= 1:
            out_cp[u - 1].wait()
            if u + 1 < SPW * 2:
                enqueue_fill((u + 1) % 2)
        pl.loop(6, SEQ_TILES // 2)(tile_pair)

        wait(ina, in_sem_a)
        plsc.parallel_loop(0, TILE_GROUPS, unroll=8)(scat_a)

        out_cp[u] = pltpu.async_copy(
            cnt, out_hbm.at[pl.ds(r0, nrows), col], out_sems[h])
    out_cp[SPW * 2 - 1].wait()


_bow_kernel = functools.partial(
    pl.kernel,
    out_type=jax.ShapeDtypeStruct((OUT_COLS, BATCH), jnp.float32),
    mesh=plsc.VectorSubcoreMesh(core_axis_name="c", subcore_axis_name="s"),
    scratch_types=[
        pltpu.VMEM((8, STRIPE), jnp.int32),
        pltpu.VMEM((8, STRIPE), jnp.int32),
        pltpu.VMEM((HALF0, STRIPE), jnp.float32),
        pltpu.VMEM((HALF1, STRIPE), jnp.float32),
        pltpu.SemaphoreType.DMA,
        pltpu.SemaphoreType.DMA,
        pltpu.SemaphoreType.DMA,
        pltpu.SemaphoreType.DMA,
        pltpu.SemaphoreType.DMA,
        pltpu.SemaphoreType.DMA,
    ],
    compiler_params=pltpu.CompilerParams(
        needs_layout_passes=False,
        use_tc_tiling_on_sc=True,
    ),
)(_bow_body)


@jax.jit
def kernel(inputs):
    zeros_hbm = jnp.zeros((HALF1, STRIPE), jnp.float32)
    out_t = _bow_kernel(inputs.T, zeros_hbm)
    return out_t.T
